# Initial kernel scaffold; baseline (speedup 1.0000x reference)
import jax
import jax.numpy as jnp
from jax.experimental import pallas as pl


def kernel(x, edge_index, edge_weight, W1, b1, Wm, bm):
    raise NotImplementedError("write your pallas kernel here")


# speedup vs baseline: 3.1406x; 3.1406x over previous
import functools

import jax
import jax.numpy as jnp
from jax import lax
from jax.experimental import pallas as pl
from jax.experimental.pallas import tpu as pltpu
from jax.experimental.pallas import tpu_sc as plsc

N = 10000
D = 128
H = 64
K = 10
E = 160000
EPS = 1e-15

_NC = 2
_NS = 16
_NW = _NC * _NS
_G = 128
_GPW = 40
_EPAD = _NW * _GPW * _G
_NPAD = 10240
_RPT = _NPAD // _NS


def _mm_body(x_ref, w_ref, o_ref):
    o_ref[...] = jnp.dot(x_ref[...], w_ref[...],
                         preferred_element_type=jnp.float32)


def _matmul(x, w, blk):
    m, d_in = x.shape
    d_out = w.shape[1]
    return pl.pallas_call(
        _mm_body,
        grid=(m // blk,),
        in_specs=[pl.BlockSpec((blk, d_in), lambda i: (i, 0)),
                  pl.BlockSpec((d_in, d_out), lambda i: (0, 0))],
        out_specs=pl.BlockSpec((blk, d_out), lambda i: (i, 0)),
        out_shape=jax.ShapeDtypeStruct((m, d_out), jnp.float32),
    )(x, w)


def _sc_scatter(h, src2d, dst2d, ew2d):
    mesh = plsc.VectorSubcoreMesh(core_axis_name="c", subcore_axis_name="s")

    @functools.partial(
        pl.kernel,
        mesh=mesh,
        out_type=jax.ShapeDtypeStruct((_NC, _NPAD, H), jnp.float32),
        scratch_types=[
            pltpu.VMEM((_GPW, _G), jnp.int32),
            pltpu.VMEM((_GPW, _G), jnp.int32),
            pltpu.VMEM((_GPW, _G), jnp.float32),
            pltpu.VMEM((_G, H), jnp.float32),
            pltpu.VMEM_SHARED((_NPAD, H), jnp.float32),
            pltpu.SemaphoreType.DMA,
        ],
        compiler_params=pltpu.CompilerParams(use_tc_tiling_on_sc=False),
    )
    def k(h_hbm, src_hbm, dst_hbm, ew_hbm, out_hbm,
          src_v, dst_v, ew_v, rows_v, acc, sem):
        c = lax.axis_index("c")
        s = lax.axis_index("s")
        wid = s * _NC + c

        zero = jnp.zeros((16,), jnp.float32)

        def zbody(r, carry):
            for q in range(H // 16):
                rows_v[r, pl.ds(16 * q, 16)] = zero
            return carry

        lax.fori_loop(0, _G, zbody, 0)
        row0 = s * _RPT
        for i in range(_RPT // _G):
            pltpu.sync_copy(rows_v, acc.at[pl.ds(row0 + _G * i, _G)])
        plsc.subcore_barrier()

        g0 = wid * _GPW
        pltpu.sync_copy(src_hbm.at[pl.ds(g0, _GPW)], src_v)
        pltpu.sync_copy(dst_hbm.at[pl.ds(g0, _GPW)], dst_v)
        pltpu.sync_copy(ew_hbm.at[pl.ds(g0, _GPW)], ew_v)

        def gbody(j, carry):
            pltpu.async_copy(h_hbm.at[src_v.at[j]], rows_v, sem).wait()

            def sbody(g, carry2):
                wv = ew_v[j, pl.ds(16 * g, 16)]
                for l in range(16):
                    e = 16 * g + l
                    w = wv[l]
                    for q in range(H // 16):
                        sl = pl.ds(16 * q, 16)
                        rows_v[e, sl] = rows_v[e, sl] * w
                return carry2

            lax.fori_loop(0, _G // 16, sbody, 0)

            pltpu.sync_copy(rows_v, acc.at[dst_v.at[j]], add=True)
            return carry

        lax.fori_loop(0, _GPW, gbody, 0)
        plsc.subcore_barrier()

        pltpu.sync_copy(acc.at[pl.ds(row0, _RPT)],
                        out_hbm.at[c].at[pl.ds(row0, _RPT)])

    return k(h, src2d, dst2d, ew2d)


def _head(parts, b1, wm, bm, blk):
    grid = N // blk

    def body(a_ref, b1_ref, wm_ref, bm_ref, p_ref, loss_ref, accsq):
        i = pl.program_id(0)

        @pl.when(i == 0)
        def _():
            accsq[...] = jnp.zeros_like(accsq)

        a = a_ref[0] + a_ref[1] + b1_ref[...]
        hr = jnp.maximum(a, 0.0)
        sblk = jnp.dot(hr, wm_ref[...],
                       preferred_element_type=jnp.float32) + bm_ref[...]
        mx = jnp.max(sblk, axis=-1, keepdims=True)
        ex = jnp.exp(sblk - mx)
        p = ex / jnp.sum(ex, axis=-1, keepdims=True)
        p_ref[...] = p
        accsq[...] += jnp.sum(p * p, axis=0, keepdims=True)

        @pl.when(i == grid - 1)
        def _():
            tr = jnp.sum(jnp.sqrt(accsq[...] + EPS))
            loss_ref[...] = jnp.reshape(-tr / jnp.sqrt(jnp.float32(N * K)),
                                        (1, 1))

    return pl.pallas_call(
        body,
        grid=(grid,),
        in_specs=[
            pl.BlockSpec((_NC, blk, H), lambda i: (0, i, 0)),
            pl.BlockSpec((1, H), lambda i: (0, 0)),
            pl.BlockSpec((H, K), lambda i: (0, 0)),
            pl.BlockSpec((1, K), lambda i: (0, 0)),
        ],
        out_specs=[
            pl.BlockSpec((blk, K), lambda i: (i, 0)),
            pl.BlockSpec((1, 1), lambda i: (0, 0)),
        ],
        out_shape=[
            jax.ShapeDtypeStruct((N, K), jnp.float32),
            jax.ShapeDtypeStruct((1, 1), jnp.float32),
        ],
        scratch_shapes=[pltpu.VMEM((1, K), jnp.float32)],
    )(parts, b1, wm, bm)


def kernel(x, edge_index, edge_weight, W1, b1, Wm, bm):
    src = edge_index[0]
    dst = edge_index[1]
    pad = _EPAD - E
    src2d = jnp.concatenate(
        [src, jnp.zeros((pad,), jnp.int32)]).reshape(_EPAD // _G, _G)
    dst2d = jnp.concatenate(
        [dst, jnp.zeros((pad,), jnp.int32)]).reshape(_EPAD // _G, _G)
    ew2d = jnp.concatenate(
        [edge_weight, jnp.zeros((pad,), jnp.float32)]).reshape(_EPAD // _G, _G)

    h = _matmul(x, W1, blk=2000)
    parts = _sc_scatter(h, src2d, dst2d, ew2d)[:, :N]
    p, loss = _head(parts, b1.reshape(1, H), Wm, bm.reshape(1, K), blk=2000)
    return p, loss.reshape(())
